# Initial kernel scaffold; baseline (speedup 1.0000x reference)
import jax
import jax.numpy as jnp
from jax.experimental import pallas as pl


def kernel(x):
    raise NotImplementedError("write your pallas kernel here")


# speedup vs baseline: 18.6416x; 18.6416x over previous
import jax
import jax.numpy as jnp
from jax.experimental import pallas as pl
from jax.experimental.pallas import tpu as pltpu

ROWS = 8192
COLS = 4096
BLOCK_ROWS = 256


def _cumsum_lanes(x):
    n = x.shape[-1]
    d = 1
    while d < n:
        shifted = jnp.concatenate(
            [jnp.zeros(x.shape[:-1] + (d,), x.dtype), x[..., :-d]], axis=-1)
        x = x + shifted
        d *= 2
    return x


def _cumsum_sublanes(x):
    n = x.shape[0]
    d = 1
    while d < n:
        shifted = jnp.concatenate(
            [jnp.zeros((d,) + x.shape[1:], x.dtype), x[:-d]], axis=0)
        x = x + shifted
        d *= 2
    return x


def _scan_block(x_ref, o_ref, carry_ref):
    i = pl.program_id(0)
    blk = x_ref[...]

    row_cs = _cumsum_lanes(blk)

    row_tot = row_cs[:, COLS - 1:COLS]
    row_incl = _cumsum_sublanes(row_tot)
    row_excl = row_incl - row_tot

    carry = jnp.where(i == 0, 0.0, carry_ref[0:1, 0:1])

    o_ref[...] = row_cs + (row_excl + carry)
    carry_ref[0:1, 0:1] = carry + row_incl[BLOCK_ROWS - 1:BLOCK_ROWS, :]


@jax.jit
def kernel(x):
    grid = ROWS // BLOCK_ROWS
    return pl.pallas_call(
        _scan_block,
        grid=(grid,),
        in_specs=[pl.BlockSpec((BLOCK_ROWS, COLS), lambda i: (i, 0))],
        out_specs=pl.BlockSpec((BLOCK_ROWS, COLS), lambda i: (i, 0)),
        out_shape=jax.ShapeDtypeStruct((ROWS, COLS), jnp.float32),
        scratch_shapes=[pltpu.VMEM((1, 1), jnp.float32)],
        compiler_params=pltpu.CompilerParams(
            dimension_semantics=("arbitrary",),
        ),
    )(x)


# speedup vs baseline: 46.8652x; 2.5140x over previous
import jax
import jax.numpy as jnp
from jax.experimental import pallas as pl
from jax.experimental.pallas import tpu as pltpu

ROWS = 8192
COLS = 4096
BLOCK_ROWS = 256
CHUNK = 128
NCHUNK = COLS // CHUNK


def _cumsum_sublanes(x):
    n = x.shape[0]
    d = 1
    while d < n:
        shifted = jnp.concatenate(
            [jnp.zeros((d,) + x.shape[1:], x.dtype), x[:-d]], axis=0)
        x = x + shifted
        d *= 2
    return x


def _scan_block(x_ref, t_ref, e_ref, p_ref, o_ref, carry_ref):
    i = pl.program_id(0)
    blk = x_ref[...]
    t = t_ref[...]

    parts = [
        jax.lax.dot(blk[:, k * CHUNK:(k + 1) * CHUNK], t,
                    preferred_element_type=jnp.float32)
        for k in range(NCHUNK)
    ]
    intra = jnp.concatenate(parts, axis=1)

    chunk_excl = jax.lax.dot(blk, e_ref[...],
                             preferred_element_type=jnp.float32)
    offs = jax.lax.dot(chunk_excl, p_ref[...],
                       preferred_element_type=jnp.float32)

    row_tot = jnp.sum(blk, axis=1, keepdims=True)
    row_incl = _cumsum_sublanes(row_tot)
    row_excl = row_incl - row_tot

    carry = jnp.where(i == 0, 0.0, carry_ref[0:1, 0:1])

    o_ref[...] = intra + offs + (row_excl + carry)
    carry_ref[0:1, 0:1] = carry + row_incl[BLOCK_ROWS - 1:BLOCK_ROWS, :]


@jax.jit
def kernel(x):
    grid = ROWS // BLOCK_ROWS

    ii = jnp.arange(CHUNK)
    t = (ii[:, None] <= ii[None, :]).astype(jnp.float32)
    cc = jnp.arange(COLS)
    kk = jnp.arange(NCHUNK)
    e = (cc[:, None] < kk[None, :] * CHUNK).astype(jnp.float32)
    p = (kk[:, None] == (jnp.arange(COLS)[None, :] // CHUNK)).astype(
        jnp.float32)

    return pl.pallas_call(
        _scan_block,
        grid=(grid,),
        in_specs=[
            pl.BlockSpec((BLOCK_ROWS, COLS), lambda i: (i, 0)),
            pl.BlockSpec((CHUNK, CHUNK), lambda i: (0, 0)),
            pl.BlockSpec((COLS, NCHUNK), lambda i: (0, 0)),
            pl.BlockSpec((NCHUNK, COLS), lambda i: (0, 0)),
        ],
        out_specs=pl.BlockSpec((BLOCK_ROWS, COLS), lambda i: (i, 0)),
        out_shape=jax.ShapeDtypeStruct((ROWS, COLS), jnp.float32),
        scratch_shapes=[pltpu.VMEM((1, 1), jnp.float32)],
        compiler_params=pltpu.CompilerParams(
            dimension_semantics=("arbitrary",),
        ),
    )(x, t, e, p)


# speedup vs baseline: 50.5881x; 1.0794x over previous
import jax
import jax.numpy as jnp
from jax.experimental import pallas as pl
from jax.experimental.pallas import tpu as pltpu

ROWS = 8192
COLS = 4096
BLOCK_ROWS = 512
CHUNK = 128
NCHUNK = COLS // CHUNK


def _cumsum_sublanes(x):
    n = x.shape[0]
    d = 1
    while d < n:
        shifted = jnp.concatenate(
            [jnp.zeros((d,) + x.shape[1:], x.dtype), x[:-d]], axis=0)
        x = x + shifted
        d *= 2
    return x


def _scan_block(x_ref, t_ref, e_ref, p_ref, o_ref, carry_ref):
    i = pl.program_id(0)
    blk = x_ref[...]
    t = t_ref[...]

    parts = [
        jax.lax.dot(blk[:, k * CHUNK:(k + 1) * CHUNK], t,
                    preferred_element_type=jnp.float32)
        for k in range(NCHUNK)
    ]
    intra = jnp.concatenate(parts, axis=1)

    chunk_excl = jax.lax.dot(blk, e_ref[...],
                             preferred_element_type=jnp.float32)
    offs = jax.lax.dot(chunk_excl, p_ref[...],
                       preferred_element_type=jnp.float32)

    row_tot = jnp.sum(blk, axis=1, keepdims=True)
    row_incl = _cumsum_sublanes(row_tot)
    row_excl = row_incl - row_tot

    carry = jnp.where(i == 0, 0.0, carry_ref[0:1, 0:1])

    o_ref[...] = intra + offs + (row_excl + carry)
    carry_ref[0:1, 0:1] = carry + row_incl[BLOCK_ROWS - 1:BLOCK_ROWS, :]


@jax.jit
def kernel(x):
    grid = ROWS // BLOCK_ROWS

    ii = jnp.arange(CHUNK)
    t = (ii[:, None] <= ii[None, :]).astype(jnp.float32)
    cc = jnp.arange(COLS)
    kk = jnp.arange(NCHUNK)
    e = (cc[:, None] < kk[None, :] * CHUNK).astype(jnp.float32)
    p = (kk[:, None] == (jnp.arange(COLS)[None, :] // CHUNK)).astype(
        jnp.float32)

    return pl.pallas_call(
        _scan_block,
        grid=(grid,),
        in_specs=[
            pl.BlockSpec((BLOCK_ROWS, COLS), lambda i: (i, 0)),
            pl.BlockSpec((CHUNK, CHUNK), lambda i: (0, 0)),
            pl.BlockSpec((COLS, NCHUNK), lambda i: (0, 0)),
            pl.BlockSpec((NCHUNK, COLS), lambda i: (0, 0)),
        ],
        out_specs=pl.BlockSpec((BLOCK_ROWS, COLS), lambda i: (i, 0)),
        out_shape=jax.ShapeDtypeStruct((ROWS, COLS), jnp.float32),
        scratch_shapes=[pltpu.VMEM((1, 1), jnp.float32)],
        compiler_params=pltpu.CompilerParams(
            dimension_semantics=("arbitrary",),
        ),
    )(x, t, e, p)
